# baseline (speedup 1.0000x reference)
import jax
import jax.numpy as jnp
from jax.experimental import pallas as pl
from jax.experimental.pallas import tpu as pltpu

_HIDDEN = 1024
_E = 8
_BT = 512


def _moe_tile(x_ref, xb_ref, gw_ref, gb_ref, wcat_ref, eb_ref, sb_ref,
              out_ref, logits_ref):
    x = x_ref[...]
    xb = xb_ref[...]

    logits = jnp.dot(x, gw_ref[...]) + gb_ref[...]
    logits_ref[...] = logits
    probs = jax.nn.softmax(logits, axis=-1)

    iota = jax.lax.broadcasted_iota(jnp.int32, probs.shape, 1)
    v1 = jnp.max(probs, axis=-1, keepdims=True)
    i1 = jnp.min(jnp.where(probs == v1, iota, _E), axis=-1, keepdims=True)
    one1 = iota == i1
    probs2 = jnp.where(one1, -jnp.inf, probs)
    v2 = jnp.max(probs2, axis=-1, keepdims=True)
    i2 = jnp.min(jnp.where(probs2 == v2, iota, _E), axis=-1, keepdims=True)
    one2 = iota == i2
    denom = v1 + v2
    combine = jnp.where(one1, v1 / denom, 0.0) + jnp.where(one2, v2 / denom, 0.0)
    combine = combine.astype(jnp.float32)

    cb = combine.astype(jnp.bfloat16)
    xcat = jnp.concatenate(
        [xb * cb[:, e:e + 1] for e in range(_E)] + [xb], axis=1)

    acc = jnp.dot(xcat, wcat_ref[...], preferred_element_type=jnp.float32)

    acc += jnp.dot(combine, eb_ref[...], preferred_element_type=jnp.float32)
    acc += sb_ref[0:1, :] + sb_ref[1:2, :]

    out_ref[...] = acc


def kernel(x, gate_w, gate_b, expert_w, expert_b, shared_w, shared_b):
    b, s, h = x.shape
    hs = x.reshape(-1, h)
    t = hs.shape[0]
    hs_bf = hs.astype(jnp.bfloat16)
    wcat = jnp.concatenate(
        [expert_w.reshape(_E * h, h),
         (shared_w[0] + shared_w[1]).reshape(h, h)], axis=0
    ).astype(jnp.bfloat16)

    grid = (t // _BT,)
    out, logits = pl.pallas_call(
        _moe_tile,
        grid=grid,
        in_specs=[
            pl.BlockSpec((_BT, h), lambda i: (i, 0)),
            pl.BlockSpec((_BT, h), lambda i: (i, 0)),
            pl.BlockSpec((h, _E), lambda i: (0, 0)),
            pl.BlockSpec((1, _E), lambda i: (0, 0)),
            pl.BlockSpec(((_E + 1) * h, h), lambda i: (0, 0)),
            pl.BlockSpec((_E, h), lambda i: (0, 0)),
            pl.BlockSpec((2, h), lambda i: (0, 0)),
        ],
        out_specs=[
            pl.BlockSpec((_BT, h), lambda i: (i, 0)),
            pl.BlockSpec((_BT, _E), lambda i: (i, 0)),
        ],
        out_shape=[
            jax.ShapeDtypeStruct((t, h), jnp.float32),
            jax.ShapeDtypeStruct((t, _E), jnp.float32),
        ],
        compiler_params=pltpu.CompilerParams(
            dimension_semantics=("arbitrary",),
        ),
    )(hs, hs_bf, gate_w, gate_b.reshape(1, _E), wcat, expert_b, shared_b)
    return out.reshape(b, s, h), logits


# speedup vs baseline: 1.1999x; 1.1999x over previous
import jax
import jax.numpy as jnp
from jax.experimental import pallas as pl
from jax.experimental.pallas import tpu as pltpu

_HIDDEN = 1024
_E = 8
_BT = 1024


def _moe_tile(x_ref, gw_ref, gb_ref, wcat_ref, eb_ref, sb_ref,
              out_ref, logits_ref):
    x = x_ref[...]

    logits = jnp.dot(x, gw_ref[...]) + gb_ref[...]
    logits_ref[...] = logits
    probs = jax.nn.softmax(logits, axis=-1)

    iota = jax.lax.broadcasted_iota(jnp.int32, probs.shape, 1)
    v1 = jnp.max(probs, axis=-1, keepdims=True)
    i1 = jnp.min(jnp.where(probs == v1, iota, _E), axis=-1, keepdims=True)
    one1 = iota == i1
    probs2 = jnp.where(one1, -jnp.inf, probs)
    v2 = jnp.max(probs2, axis=-1, keepdims=True)
    i2 = jnp.min(jnp.where(probs2 == v2, iota, _E), axis=-1, keepdims=True)
    one2 = iota == i2
    denom = v1 + v2
    combine = jnp.where(one1, v1 / denom, 0.0) + jnp.where(one2, v2 / denom, 0.0)
    combine = combine.astype(jnp.float32)

    acc = jnp.dot(combine, eb_ref[...], preferred_element_type=jnp.float32)
    acc += sb_ref[0:1, :] + sb_ref[1:2, :]

    xb = x.astype(jnp.bfloat16)
    acc += jnp.dot(xb, wcat_ref[_E], preferred_element_type=jnp.float32)
    for e in range(_E):
        xe = (x * combine[:, e:e + 1]).astype(jnp.bfloat16)
        acc += jnp.dot(xe, wcat_ref[e], preferred_element_type=jnp.float32)

    out_ref[...] = acc


def kernel(x, gate_w, gate_b, expert_w, expert_b, shared_w, shared_b):
    b, s, h = x.shape
    hs = x.reshape(-1, h)
    t = hs.shape[0]
    wcat = jnp.concatenate(
        [expert_w, (shared_w[0] + shared_w[1])[None]], axis=0
    ).astype(jnp.bfloat16)

    grid = (t // _BT,)
    out, logits = pl.pallas_call(
        _moe_tile,
        grid=grid,
        in_specs=[
            pl.BlockSpec((_BT, h), lambda i: (i, 0)),
            pl.BlockSpec((h, _E), lambda i: (0, 0)),
            pl.BlockSpec((1, _E), lambda i: (0, 0)),
            pl.BlockSpec((_E + 1, h, h), lambda i: (0, 0, 0)),
            pl.BlockSpec((_E, h), lambda i: (0, 0)),
            pl.BlockSpec((2, h), lambda i: (0, 0)),
        ],
        out_specs=[
            pl.BlockSpec((_BT, h), lambda i: (i, 0)),
            pl.BlockSpec((_BT, _E), lambda i: (i, 0)),
        ],
        out_shape=[
            jax.ShapeDtypeStruct((t, h), jnp.float32),
            jax.ShapeDtypeStruct((t, _E), jnp.float32),
        ],
        compiler_params=pltpu.CompilerParams(
            dimension_semantics=("arbitrary",),
        ),
    )(hs, gate_w, gate_b.reshape(1, _E), wcat, expert_b, shared_b)
    return out.reshape(b, s, h), logits
